# Initial kernel scaffold; baseline (speedup 1.0000x reference)
import jax
import jax.numpy as jnp
from jax.experimental import pallas as pl


def kernel(iwords, owords, ivectors, ovectors):
    raise NotImplementedError("write your pallas kernel here")


# speedup vs baseline: 4.2384x; 4.2384x over previous
import functools

import jax
import jax.numpy as jnp
from jax import lax
from jax.experimental import pallas as pl
from jax.experimental.pallas import tpu as pltpu
from jax.experimental.pallas import tpu_sc as plsc

V = 1000000
D = 64
NNEG = 5
B = 16384
C = 20
J = C * (1 + NNEG)
JP = 128

NW = 32
BPW = B // NW
NB = 4
NCHUNK = BPW // NB


def _sc_scores_body(iw_hbm, idx_hbm, ivec_hbm, ovec_hbm, out_hbm,
                    iw_v, iv_all, idx_cv, rows_v, scores_cv, sem):
    cid = lax.axis_index("c")
    sid = lax.axis_index("s")
    wid = sid * 2 + cid
    b0 = wid * BPW

    iota = lax.iota(jnp.int32, 16)

    pltpu.sync_copy(iw_hbm.at[pl.ds(wid * 4, 4)], iw_v)
    handles = [
        pltpu.async_copy(ivec_hbm.at[iw_v.at[i]],
                         iv_all.at[pl.ds(i * 128, 128)], sem)
        for i in range(4)
    ]
    for h in handles:
        h.wait()

    def chunk_body(c, carry):
        base = b0 + c * NB
        pltpu.sync_copy(idx_hbm.at[pl.ds(base, NB)], idx_cv)
        hs = [
            pltpu.async_copy(ovec_hbm.at[idx_cv.at[i]],
                             rows_v.at[pl.ds(i * JP, JP)], sem)
            for i in range(NB)
        ]
        for h in hs:
            h.wait()
        for bi in range(NB):
            brow = c * NB + bi
            ivvs = [iv_all[brow, pl.ds(q * 16, 16)] for q in range(4)]

            def g_body(g, carry2, bi=bi, ivvs=ivvs):
                acc = jnp.zeros((16,), jnp.float32)
                for u in range(16):
                    r = bi * JP + g * 16 + u
                    p = (rows_v[r, pl.ds(0, 16)] * ivvs[0]
                         + rows_v[r, pl.ds(16, 16)] * ivvs[1]
                         + rows_v[r, pl.ds(32, 16)] * ivvs[2]
                         + rows_v[r, pl.ds(48, 16)] * ivvs[3])
                    acc = jnp.where(iota == u, jnp.sum(p), acc)
                scores_cv[bi, pl.ds(g * 16, 16)] = acc
                return carry2

            lax.fori_loop(0, JP // 16, g_body, 0)
        pltpu.sync_copy(scores_cv, out_hbm.at[pl.ds(base, NB)])
        return carry

    lax.fori_loop(0, NCHUNK, chunk_body, 0)


@jax.jit
def _sc_scores(iw2d, idx, ivectors, ovectors):
    mesh = plsc.VectorSubcoreMesh(core_axis_name="c", subcore_axis_name="s")
    return pl.kernel(
        _sc_scores_body,
        mesh=mesh,
        compiler_params=pltpu.CompilerParams(
            needs_layout_passes=False, use_tc_tiling_on_sc=False),
        out_type=jax.ShapeDtypeStruct((B, JP), jnp.float32),
        scratch_types=[
            pltpu.VMEM((4, 128), jnp.int32),
            pltpu.VMEM((BPW, D), jnp.float32),
            pltpu.VMEM((NB, JP), jnp.int32),
            pltpu.VMEM((NB * JP, D), jnp.float32),
            pltpu.VMEM((NB, JP), jnp.float32),
            pltpu.SemaphoreType.DMA,
        ],
    )(iw2d, idx, ivectors, ovectors)


def _tc_reduce_body(s_ref, o_ref):
    i = pl.program_id(0)
    x = s_ref[...]
    col = lax.broadcasted_iota(jnp.int32, x.shape, 1)
    z = jnp.where(col < C, x, -x)
    ls = jnp.minimum(z, 0.0) - jnp.log1p(jnp.exp(-jnp.abs(z)))
    ls = jnp.where(col < J, ls, 0.0)
    psum = jnp.sum(ls)

    @pl.when(i == 0)
    def _():
        o_ref[0, 0] = 0.0

    o_ref[0, 0] += psum

    @pl.when(i == pl.num_programs(0) - 1)
    def _():
        o_ref[0, 0] = o_ref[0, 0] * (-1.0 / (B * C))


@jax.jit
def _tc_reduce(scores):
    rows = 1024
    out = pl.pallas_call(
        _tc_reduce_body,
        grid=(B // rows,),
        in_specs=[pl.BlockSpec((rows, JP), lambda i: (i, 0))],
        out_specs=pl.BlockSpec(memory_space=pltpu.SMEM),
        out_shape=jax.ShapeDtypeStruct((1, 1), jnp.float32),
    )(scores)
    return out[0, 0]


def kernel(iwords, owords, ivectors, ovectors):
    nkey = jax.random.key(12345)
    nwords = jax.random.randint(nkey, (B, C * NNEG), 0, V - 1).astype(jnp.int32)
    pad = jnp.zeros((B, JP - J), jnp.int32)
    idx = jnp.concatenate([owords, nwords, pad], axis=1)
    iw2d = iwords.reshape(B // 128, 128)
    scores = _sc_scores(iw2d, idx, ivectors, ovectors)
    return _tc_reduce(scores)
